# baseline (speedup 1.0000x reference)
import functools

import jax
import jax.numpy as jnp
from jax import lax
from jax.experimental import pallas as pl
from jax.experimental.pallas import tpu as pltpu
from jax.experimental.pallas import tpu_sc as plsc

_VB = 1024
_UP = 16
_RG = 4
_NS = 8


def _argmax_step(x_ref, s, rmax_ref, ridx_ref):
    xb = x_ref[0]
    bmax = jnp.max(xb, axis=0, keepdims=True)
    iota = lax.broadcasted_iota(jnp.int32, xb.shape, 0)
    bidx = jnp.min(jnp.where(xb == bmax, iota, jnp.int32(2**30)),
                   axis=0, keepdims=True) + s * _VB

    @pl.when(s == 0)
    def _():
        rmax_ref[...] = bmax
        ridx_ref[...] = bidx

    @pl.when(s > 0)
    def _():
        better = bmax > rmax_ref[...]
        ridx_ref[...] = jnp.where(better, bidx, ridx_ref[...])
        rmax_ref[...] = jnp.where(better, bmax, rmax_ref[...])


def _argmax_body(x_ref, idx_ref, rmax_ref, ridx_ref):
    s = pl.program_id(0)
    _argmax_step(x_ref, s, rmax_ref, ridx_ref)

    @pl.when(s == _NS - 1)
    def _():
        idx_ref[0] = ridx_ref[...]


def _sc_gather(table_hbm, idx_hbm, out_hbm, idx_v, rows_v, sem, *, b_per_w):
    wid = lax.axis_index("s") * 2 + lax.axis_index("c")
    base = wid * b_per_w
    pltpu.sync_copy(idx_hbm.at[pl.ds(base, b_per_w)], idx_v)
    pltpu.async_copy(table_hbm.at[idx_v], rows_v, sem).wait()
    pltpu.sync_copy(rows_v, out_hbm.at[pl.ds(base, b_per_w)])


def _decode_step(emb_ref, wd_ref, bd_ref, w1_ref, b1_ref, w2_ref, b2_ref,
                 w3_ref, b3_ref, e_ref, logits_ref, seg_ref, *, nc, wg):
    for i in range(_RG):
        emb = emb_ref[i * wg:(i + 1) * wg, :]
        dec = emb @ wd_ref[...] + bd_ref[...]
        sig = jax.nn.sigmoid(dec)
        h = jax.nn.relu(sig @ w1_ref[...] + b1_ref[...])
        h = jax.nn.relu(h @ w2_ref[...] + b2_ref[...])
        lg = h @ w3_ref[...] + b3_ref[...]

        lg_w = lax.dot_general(lg, e_ref[...], (((0,), (0,)), ((), ())),
                               preferred_element_type=jnp.float32)
        sg_w = lax.dot_general(sig, e_ref[...], (((0,), (0,)), ((), ())),
                               preferred_element_type=jnp.float32)
        logits_ref[0, :, i * _UP:(i + 1) * _UP, :] = jnp.broadcast_to(
            lg_w[:, None, :], (nc, _UP, lg_w.shape[1]))
        seg_ref[0, :, i * _UP:(i + 1) * _UP, :] = jnp.broadcast_to(
            sg_w[:, None, :], (3, _UP, sg_w.shape[1]))


def _fused_body(x_ref, emb_ref, wd_ref, bd_ref, w1_ref, b1_ref,
                w2_ref, b2_ref, w3_ref, b3_ref, e_ref,
                logits_ref, seg_ref, idx_ref, rmax_ref, ridx_ref,
                *, nc, wg):
    s = pl.program_id(0)
    _decode_step(emb_ref, wd_ref, bd_ref, w1_ref, b1_ref, w2_ref, b2_ref,
                 w3_ref, b3_ref, e_ref, logits_ref, seg_ref, nc=nc, wg=wg)
    _argmax_step(x_ref, s, rmax_ref, ridx_ref)

    @pl.when(s == _NS - 1)
    def _():
        idx_ref[0] = ridx_ref[...]


def _decode_body(emb_ref, wd_ref, bd_ref, w1_ref, b1_ref,
                 w2_ref, b2_ref, w3_ref, b3_ref, e_ref,
                 lg_in_ref, sg_in_ref, logits_ref, seg_ref, *, nc, wg):
    del lg_in_ref, sg_in_ref
    _decode_step(emb_ref, wd_ref, bd_ref, w1_ref, b1_ref, w2_ref, b2_ref,
                 w3_ref, b3_ref, e_ref, logits_ref, seg_ref, nc=nc, wg=wg)


def kernel(x, codebook, W_dec, b_dec, W1, b1, W2, b2, W3, b3):
    B, V, Hg, Wg = x.shape
    P = Hg * Wg
    D = codebook.shape[1]
    NC = W3.shape[1]
    H, W = Hg * _UP, Wg * _UP
    xr = x.reshape(B, V, P)
    cells = _RG * Wg

    info = plsc.get_sparse_core_info()
    nw = info.num_cores * info.num_subcores
    b_per_w = P // nw
    mesh = plsc.VectorSubcoreMesh(core_axis_name="c", subcore_axis_name="s")

    def sc_gather(idx_b):
        return functools.partial(
            pl.kernel,
            mesh=mesh,
            out_type=jax.ShapeDtypeStruct((P, D), jnp.float32),
            scratch_types=[pltpu.VMEM((b_per_w,), jnp.int32),
                           pltpu.VMEM((b_per_w, D), jnp.float32),
                           pltpu.SemaphoreType.DMA],
        )(functools.partial(_sc_gather, b_per_w=b_per_w))(codebook, idx_b)

    E = (jnp.arange(W, dtype=jnp.int32)[None, :] // _UP
         == jnp.arange(Wg, dtype=jnp.int32)[:, None]).astype(jnp.float32)
    wargs = (W_dec, b_dec.reshape(1, 3), W1, b1.reshape(1, 32),
             W2, b2.reshape(1, 32), W3, b3.reshape(1, NC), E)
    wspecs = [pl.BlockSpec(s, lambda s_: (0,) * len(s)) for s in
              [(D, 3), (1, 3), (3, 32), (1, 32), (32, 32), (1, 32),
               (32, NC), (1, NC), (Wg, W)]]

    idx0 = pl.pallas_call(
        _argmax_body,
        grid=(_NS,),
        in_specs=[pl.BlockSpec((1, _VB, P), lambda s: (0, s, 0))],
        out_specs=pl.BlockSpec((1, 1, P), lambda s: (0, 0, 0)),
        out_shape=jax.ShapeDtypeStruct((1, 1, P), jnp.int32),
        scratch_shapes=[pltpu.VMEM((1, P), jnp.float32),
                        pltpu.VMEM((1, P), jnp.int32)],
    )(xr)

    emb0 = sc_gather(idx0.reshape(P))

    logits, seg, idx1 = pl.pallas_call(
        functools.partial(_fused_body, nc=NC, wg=Wg),
        grid=(_NS,),
        in_specs=[pl.BlockSpec((1, _VB, P), lambda s: (1, s, 0)),
                  pl.BlockSpec((cells, D), lambda s: (s, 0))] + wspecs,
        out_specs=[
            pl.BlockSpec((1, NC, _RG * _UP, W), lambda s: (0, 0, s, 0)),
            pl.BlockSpec((1, 3, _RG * _UP, W), lambda s: (0, 0, s, 0)),
            pl.BlockSpec((1, 1, P), lambda s: (0, 0, 0)),
        ],
        out_shape=[jax.ShapeDtypeStruct((B, NC, H, W), jnp.float32),
                   jax.ShapeDtypeStruct((B, 3, H, W), jnp.float32),
                   jax.ShapeDtypeStruct((1, 1, P), jnp.int32)],
        scratch_shapes=[pltpu.VMEM((1, P), jnp.float32),
                        pltpu.VMEM((1, P), jnp.int32)],
    )(xr, emb0, *wargs)

    emb1 = sc_gather(idx1.reshape(P))

    nin = 1 + len(wargs)
    logits, seg = pl.pallas_call(
        functools.partial(_decode_body, nc=NC, wg=Wg),
        grid=(_NS,),
        in_specs=[pl.BlockSpec((cells, D), lambda s: (s, 0))] + wspecs + [
            pl.BlockSpec((1, 1, 8, 128), lambda s: (0, 0, 0, 0)),
            pl.BlockSpec((1, 1, 8, 128), lambda s: (0, 0, 0, 0)),
        ],
        out_specs=[
            pl.BlockSpec((1, NC, _RG * _UP, W), lambda s: (1, 0, s, 0)),
            pl.BlockSpec((1, 3, _RG * _UP, W), lambda s: (1, 0, s, 0)),
        ],
        out_shape=[jax.ShapeDtypeStruct((B, NC, H, W), jnp.float32),
                   jax.ShapeDtypeStruct((B, 3, H, W), jnp.float32)],
        input_output_aliases={nin: 0, nin + 1: 1},
    )(emb1, *wargs, logits, seg)

    return logits, seg


# speedup vs baseline: 1.0540x; 1.0540x over previous
import functools

import jax
import jax.numpy as jnp
from jax import lax
from jax.experimental import pallas as pl
from jax.experimental.pallas import tpu as pltpu
from jax.experimental.pallas import tpu_sc as plsc

_VB = 2048
_UP = 16
_RG = 4


def _argmax_body(x_ref, out_ref, rmax_ref, ridx_ref, *, num_vb):
    k = pl.program_id(1)
    xb = x_ref[0]
    bmax = jnp.max(xb, axis=0, keepdims=True)
    iota = lax.broadcasted_iota(jnp.int32, xb.shape, 0)
    bidx = jnp.min(jnp.where(xb == bmax, iota, jnp.int32(2**30)),
                   axis=0, keepdims=True) + k * _VB

    @pl.when(k == 0)
    def _():
        rmax_ref[...] = bmax
        ridx_ref[...] = bidx

    @pl.when(k > 0)
    def _():
        better = bmax > rmax_ref[...]
        ridx_ref[...] = jnp.where(better, bidx, ridx_ref[...])
        rmax_ref[...] = jnp.where(better, bmax, rmax_ref[...])

    @pl.when(k == num_vb - 1)
    def _():
        out_ref[0] = ridx_ref[...]


def _sc_gather(table_hbm, idx_hbm, out_hbm, idx_v, rows_v, sem, *, b_per_w):
    wid = lax.axis_index("s") * 2 + lax.axis_index("c")
    base = wid * b_per_w
    pltpu.sync_copy(idx_hbm.at[pl.ds(base, b_per_w)], idx_v)
    pltpu.async_copy(table_hbm.at[idx_v], rows_v, sem).wait()
    pltpu.sync_copy(rows_v, out_hbm.at[pl.ds(base, b_per_w)])


def _decode_body(emb_ref, wd_ref, bd_ref, w1_ref, b1_ref,
                 w2_ref, b2_ref, w3_ref, b3_ref, e_ref,
                 logits_ref, seg_ref, *, nc, wg):
    for i in range(_RG):
        emb = emb_ref[0, i * wg:(i + 1) * wg, :]
        dec = emb @ wd_ref[...] + bd_ref[...]
        sig = jax.nn.sigmoid(dec)
        h = jax.nn.relu(sig @ w1_ref[...] + b1_ref[...])
        h = jax.nn.relu(h @ w2_ref[...] + b2_ref[...])
        lg = h @ w3_ref[...] + b3_ref[...]

        lg_w = lax.dot_general(lg, e_ref[...], (((0,), (0,)), ((), ())),
                               preferred_element_type=jnp.float32)
        sg_w = lax.dot_general(sig, e_ref[...], (((0,), (0,)), ((), ())),
                               preferred_element_type=jnp.float32)
        logits_ref[0, :, i * _UP:(i + 1) * _UP, :] = jnp.broadcast_to(
            lg_w[:, None, :], (nc, _UP, lg_w.shape[1]))
        seg_ref[0, :, i * _UP:(i + 1) * _UP, :] = jnp.broadcast_to(
            sg_w[:, None, :], (3, _UP, sg_w.shape[1]))


def kernel(x, codebook, W_dec, b_dec, W1, b1, W2, b2, W3, b3):
    B, V, Hg, Wg = x.shape
    P = Hg * Wg
    D = codebook.shape[1]
    NC = W3.shape[1]
    H, W = Hg * _UP, Wg * _UP
    num_vb = V // _VB

    idx = pl.pallas_call(
        functools.partial(_argmax_body, num_vb=num_vb),
        grid=(B, num_vb),
        in_specs=[pl.BlockSpec((1, _VB, P), lambda b, k: (b, k, 0))],
        out_specs=pl.BlockSpec((1, 1, P), lambda b, k: (b, 0, 0)),
        out_shape=jax.ShapeDtypeStruct((B, 1, P), jnp.int32),
        scratch_shapes=[pltpu.VMEM((1, P), jnp.float32),
                        pltpu.VMEM((1, P), jnp.int32)],
    )(x.reshape(B, V, P))

    info = plsc.get_sparse_core_info()
    nw = info.num_cores * info.num_subcores
    b_per_w = (B * P) // nw
    mesh = plsc.VectorSubcoreMesh(core_axis_name="c", subcore_axis_name="s")
    emb = functools.partial(
        pl.kernel,
        mesh=mesh,
        out_type=jax.ShapeDtypeStruct((B * P, D), jnp.float32),
        scratch_types=[pltpu.VMEM((b_per_w,), jnp.int32),
                       pltpu.VMEM((b_per_w, D), jnp.float32),
                       pltpu.SemaphoreType.DMA],
    )(functools.partial(_sc_gather, b_per_w=b_per_w))(
        codebook, idx.reshape(B * P))

    E = (jnp.arange(W, dtype=jnp.int32)[None, :] // _UP
         == jnp.arange(Wg, dtype=jnp.int32)[:, None]).astype(jnp.float32)

    full = lambda shape: pl.BlockSpec(shape, lambda b, hg: (0,) * len(shape))
    logits, seg = pl.pallas_call(
        functools.partial(_decode_body, nc=NC, wg=Wg),
        grid=(B, Hg // _RG),
        in_specs=[
            pl.BlockSpec((1, _RG * Wg, D), lambda b, hg: (b, hg, 0)),
            full((D, 3)), full((1, 3)),
            full((3, 32)), full((1, 32)),
            full((32, 32)), full((1, 32)),
            full((32, NC)), full((1, NC)),
            full((Wg, W)),
        ],
        out_specs=[
            pl.BlockSpec((1, NC, _RG * _UP, W), lambda b, hg: (b, 0, hg, 0)),
            pl.BlockSpec((1, 3, _RG * _UP, W), lambda b, hg: (b, 0, hg, 0)),
        ],
        out_shape=[jax.ShapeDtypeStruct((B, NC, H, W), jnp.float32),
                   jax.ShapeDtypeStruct((B, 3, H, W), jnp.float32)],
    )(emb.reshape(B, P, D), W_dec, b_dec.reshape(1, 3), W1, b1.reshape(1, 32),
      W2, b2.reshape(1, 32), W3, b3.reshape(1, NC), E)

    return logits, seg
